# baseline (speedup 1.0000x reference)
import functools

import jax
import jax.numpy as jnp
from jax import lax
from jax.experimental import pallas as pl
from jax.experimental.pallas import tpu as pltpu
from jax.experimental.pallas import tpu_sc as plsc

_VOCAB = 32000
_D = 768
_E = 8
_T = 2048
_BM = 256
_NPAD = _T * 2 + _E * _BM
_NB = _NPAD // _BM


def _make_row_gather(D, B, dtype=jnp.float32):
    info = plsc.get_sparse_core_info()
    NW = info.num_cores * info.num_subcores
    NC = info.num_cores
    assert B % NW == 0
    b_per_w = B // NW
    mesh = plsc.VectorSubcoreMesh(core_axis_name="c", subcore_axis_name="s")

    def k(table_hbm, idx_hbm, out_hbm, idx_v, rows_v, sem):
        wid = lax.axis_index("s") * NC + lax.axis_index("c")
        base = wid * b_per_w
        pltpu.sync_copy(idx_hbm.at[pl.ds(base, b_per_w)], idx_v)
        pltpu.async_copy(table_hbm.at[idx_v], rows_v, sem).wait()
        pltpu.sync_copy(rows_v, out_hbm.at[pl.ds(base, b_per_w)])

    def run(table, idx):
        return pl.kernel(
            k, mesh=mesh,
            out_type=jax.ShapeDtypeStruct((B, D), dtype),
            scratch_types=[
                pltpu.VMEM((b_per_w,), jnp.int32),
                pltpu.VMEM((b_per_w, D), dtype),
                pltpu.SemaphoreType.DMA,
            ],
        )(table, idx)

    return run


def _make_row_scatter(D, B, OUT_ROWS, dtype=jnp.float32):
    info = plsc.get_sparse_core_info()
    NW = info.num_cores * info.num_subcores
    NC = info.num_cores
    assert B % NW == 0
    b_per_w = B // NW
    mesh = plsc.VectorSubcoreMesh(core_axis_name="c", subcore_axis_name="s")

    def k(rows_hbm, idx_hbm, out_hbm, idx0_v, idx1_v, rows_v, sem):
        wid = lax.axis_index("s") * NC + lax.axis_index("c")
        base = wid * b_per_w
        pltpu.sync_copy(rows_hbm.at[pl.ds(base, b_per_w)], rows_v)
        pltpu.sync_copy(idx_hbm.at[pl.ds(base, b_per_w)], idx0_v)
        pltpu.sync_copy(idx_hbm.at[pl.ds(B + base, b_per_w)], idx1_v)
        c0 = pltpu.async_copy(rows_v, out_hbm.at[idx0_v], sem)
        c1 = pltpu.async_copy(rows_v, out_hbm.at[idx1_v], sem)
        c0.wait()
        c1.wait()

    def run(rows, idx):
        return pl.kernel(
            k, mesh=mesh,
            out_type=jax.ShapeDtypeStruct((OUT_ROWS, D), dtype),
            scratch_types=[
                pltpu.VMEM((b_per_w,), jnp.int32),
                pltpu.VMEM((b_per_w,), jnp.int32),
                pltpu.VMEM((b_per_w, D), dtype),
                pltpu.SemaphoreType.DMA,
            ],
        )(rows, idx)

    return run


def _routing_math(h, wg):
    logits = jnp.dot(h, wg, preferred_element_type=jnp.float32)

    eiota = lax.broadcasted_iota(jnp.int32, logits.shape, 1)
    v0 = jnp.max(logits, axis=-1, keepdims=True)
    i0 = jnp.min(jnp.where(logits == v0, eiota, _E), axis=-1, keepdims=True)
    masked = jnp.where(eiota == i0, -jnp.inf, logits)
    v1 = jnp.max(masked, axis=-1, keepdims=True)
    i1 = jnp.min(jnp.where(masked == v1, eiota, _E), axis=-1, keepdims=True)

    ex1 = jnp.exp(v1 - v0)
    w0 = 1.0 / (1.0 + ex1)
    w1 = ex1 / (1.0 + ex1)
    wts = jnp.concatenate([w0, w1], axis=1)

    one0 = (eiota == i0).astype(jnp.bfloat16)
    one1 = (eiota == i1).astype(jnp.bfloat16)

    T = h.shape[0]
    r_iota = lax.broadcasted_iota(jnp.int32, (T, T), 0)
    c_iota = lax.broadcasted_iota(jnp.int32, (T, T), 1)
    tril = (c_iota < r_iota).astype(jnp.bfloat16)

    cum0 = jnp.dot(tril, one0, preferred_element_type=jnp.float32)
    cum1 = jnp.dot(tril, one1, preferred_element_type=jnp.float32)
    tot0 = jnp.sum(one0.astype(jnp.float32), axis=0, keepdims=True)
    tot1 = jnp.sum(one1.astype(jnp.float32), axis=0, keepdims=True)
    counts = tot0 + tot1

    pc = jnp.ceil(counts * (1.0 / _BM)) * _BM
    e_r = lax.broadcasted_iota(jnp.int32, (_E, _E), 0)
    e_c = lax.broadcasted_iota(jnp.int32, (_E, _E), 1)
    m8 = (e_r < e_c).astype(jnp.float32)
    po = jnp.dot(pc, m8, preferred_element_type=jnp.float32)

    rank0 = jnp.sum(one0.astype(jnp.float32) * (cum0 + po), axis=1, keepdims=True)
    rank1 = jnp.sum(one1.astype(jnp.float32) * (cum1 + tot0 + po), axis=1,
                    keepdims=True)
    dest = jnp.concatenate(
        [rank0, rank1], axis=1).astype(jnp.int32)

    pend = po + pc
    bstart = (lax.broadcasted_iota(jnp.int32, (_NB, _E), 0) * _BM).astype(
        jnp.float32)
    be = jnp.sum((jnp.broadcast_to(pend, (_NB, _E)) <= bstart).astype(
        jnp.int32), axis=1, keepdims=True)
    return dest, wts, jnp.minimum(be, _E - 1)


def _router_body(h_ref, wg_ref, dest_ref, wts_ref, be_ref):
    dest, wts, be = _routing_math(h_ref[...], wg_ref[...])
    dest_ref[...] = dest
    wts_ref[...] = wts
    be_ref[...] = be


def _router(h, Wg):
    return pl.pallas_call(
        _router_body,
        in_specs=[
            pl.BlockSpec((_T, _D), lambda: (0, 0)),
            pl.BlockSpec((_D, _E), lambda: (0, 0)),
        ],
        out_specs=[
            pl.BlockSpec((_T, 2), lambda: (0, 0)),
            pl.BlockSpec((_T, 2), lambda: (0, 0)),
            pl.BlockSpec((_NB, 1), lambda: (0, 0)),
        ],
        out_shape=[
            jax.ShapeDtypeStruct((_T, 2), jnp.int32),
            jax.ShapeDtypeStruct((_T, 2), jnp.float32),
            jax.ShapeDtypeStruct((_NB, 1), jnp.int32),
        ],
    )(h, Wg)


def _combine_router_body(a_ref, wts_ref, wg_ref, y_ref, dest_ref, wts2_ref,
                         be_ref):
    a0 = a_ref[0:_T, :].astype(jnp.float32)
    a1 = a_ref[_T:2 * _T, :].astype(jnp.float32)
    w = wts_ref[...].astype(jnp.bfloat16).astype(jnp.float32)
    y = (w[:, 0:1] * a0 + w[:, 1:2] * a1).astype(jnp.bfloat16)
    y_ref[...] = y
    dest, wts2, be = _routing_math(y, wg_ref[...])
    dest_ref[...] = dest
    wts2_ref[...] = wts2
    be_ref[...] = be


def _combine_router(A, wts, Wg):
    return pl.pallas_call(
        _combine_router_body,
        in_specs=[
            pl.BlockSpec((2 * _T, _D), lambda: (0, 0)),
            pl.BlockSpec((_T, 2), lambda: (0, 0)),
            pl.BlockSpec((_D, _E), lambda: (0, 0)),
        ],
        out_specs=[
            pl.BlockSpec((_T, _D), lambda: (0, 0)),
            pl.BlockSpec((_T, 2), lambda: (0, 0)),
            pl.BlockSpec((_T, 2), lambda: (0, 0)),
            pl.BlockSpec((_NB, 1), lambda: (0, 0)),
        ],
        out_shape=[
            jax.ShapeDtypeStruct((_T, _D), jnp.bfloat16),
            jax.ShapeDtypeStruct((_T, 2), jnp.int32),
            jax.ShapeDtypeStruct((_T, 2), jnp.float32),
            jax.ShapeDtypeStruct((_NB, 1), jnp.int32),
        ],
    )(A, wts, Wg)


def _grouped_body(be_ref, x_ref, w1_ref, b1_ref, w2_ref, b2_ref, o_ref):
    x = x_ref[...]
    hid = jnp.maximum(
        jnp.dot(x, w1_ref[0], preferred_element_type=jnp.float32) + b1_ref[0],
        0.0).astype(jnp.bfloat16)
    out = (jnp.dot(hid, w2_ref[0], preferred_element_type=jnp.float32)
           + b2_ref[0]).astype(jnp.bfloat16)
    o_ref[...] = out


def _grouped(Xs, W1, b1, W2, b2, be):
    grid_spec = pltpu.PrefetchScalarGridSpec(
        num_scalar_prefetch=1,
        grid=(_NB,),
        in_specs=[
            pl.BlockSpec((_BM, _D), lambda b, be: (b, 0)),
            pl.BlockSpec((1, _D, _D), lambda b, be: (be[b], 0, 0)),
            pl.BlockSpec((1, 1, _D), lambda b, be: (be[b], 0, 0)),
            pl.BlockSpec((1, _D, _D), lambda b, be: (be[b], 0, 0)),
            pl.BlockSpec((1, 1, _D), lambda b, be: (be[b], 0, 0)),
        ],
        out_specs=pl.BlockSpec((_BM, _D), lambda b, be: (b, 0)),
    )
    return pl.pallas_call(
        _grouped_body,
        grid_spec=grid_spec,
        out_shape=jax.ShapeDtypeStruct((_NPAD, _D), jnp.bfloat16),
    )(be, Xs, W1, b1.reshape(_E, 1, _D), W2, b2.reshape(_E, 1, _D))


def _combine_body(a_ref, wts_ref, o_ref):
    a0 = a_ref[0:_T, :].astype(jnp.float32)
    a1 = a_ref[_T:2 * _T, :].astype(jnp.float32)
    w = wts_ref[...].astype(jnp.bfloat16).astype(jnp.float32)
    y = w[:, 0:1] * a0 + w[:, 1:2] * a1
    o_ref[...] = y.astype(jnp.bfloat16)


def _combine(A, wts):
    return pl.pallas_call(
        _combine_body,
        in_specs=[
            pl.BlockSpec((2 * _T, _D), lambda: (0, 0)),
            pl.BlockSpec((_T, 2), lambda: (0, 0)),
        ],
        out_specs=pl.BlockSpec((_T, _D), lambda: (0, 0)),
        out_shape=jax.ShapeDtypeStruct((_T, _D), jnp.bfloat16),
    )(A, wts)


def _proj_body(h_ref, w_ref, b_ref, out_ref):
    out_ref[...] = (
        jnp.dot(h_ref[...], w_ref[...], preferred_element_type=jnp.float32)
        + b_ref[...]
    )


def _proj(h, Wout, bout2d, bn=640):
    T, D = h.shape
    V = Wout.shape[1]
    grid = (V // bn,)
    return pl.pallas_call(
        _proj_body,
        grid=grid,
        in_specs=[
            pl.BlockSpec((T, D), lambda n: (0, 0)),
            pl.BlockSpec((D, bn), lambda n: (0, n)),
            pl.BlockSpec((1, bn), lambda n: (0, n)),
        ],
        out_specs=pl.BlockSpec((T, bn), lambda n: (0, n)),
        out_shape=jax.ShapeDtypeStruct((T, V), jnp.float32),
    )(h, Wout, bout2d)


def _as_i32(x_bf):
    n = x_bf.shape[0]
    return lax.bitcast_convert_type(
        x_bf.reshape(n, _D // 2, 2), jnp.int32)


def _as_bf(x_i32):
    n = x_i32.shape[0]
    return lax.bitcast_convert_type(
        x_i32, jnp.bfloat16).reshape(n, _D)


def _dispatch_ffn(h_bf, dest, be, W1, b1, W2, b2):
    dest_flat = jnp.concatenate([dest[:, 0], dest[:, 1]])
    Xs = _make_row_scatter(_D // 2, _T, _NPAD, jnp.int32)(
        _as_i32(h_bf), dest_flat)
    out_s = _grouped(_as_bf(Xs), W1, b1, W2, b2, be.reshape(_NB))
    A = _make_row_gather(_D // 2, 2 * _T, jnp.int32)(
        _as_i32(out_s), dest_flat)
    return _as_bf(A)


def kernel(x, emb, Wg1, W1a, b1a, W2a, b2a, Wg2, W1b, b1b, W2b, b2b, Wout, bout):
    B, S = x.shape
    bf = jnp.bfloat16
    idx = x.reshape(-1).astype(jnp.int32)
    h32 = _make_row_gather(_D, _T)(emb, idx)
    h_bf = h32.astype(bf)

    dest1, wts1, be1 = _router(h_bf, Wg1.astype(bf))
    A1 = _dispatch_ffn(h_bf, dest1, be1, W1a.astype(bf), b1a,
                       W2a.astype(bf), b2a)
    y1, dest2, wts2, be2 = _combine_router(A1, wts1, Wg2.astype(bf))
    A2 = _dispatch_ffn(y1, dest2, be2, W1b.astype(bf), b1b,
                       W2b.astype(bf), b2b)
    h2 = _combine(A2, wts2)
    out = _proj(h2, Wout.astype(bf), bout.reshape(1, -1))
    return out.reshape(B, S, _VOCAB)


# speedup vs baseline: 2.6892x; 2.6892x over previous
import functools

import jax
import jax.numpy as jnp
from jax import lax
from jax.experimental import pallas as pl
from jax.experimental.pallas import tpu as pltpu
from jax.experimental.pallas import tpu_sc as plsc

_VOCAB = 32000
_D = 768
_E = 8
_T = 2048
_BM = 256
_NPAD = _T * 2 + _E * _BM
_NB = _NPAD // _BM


def _make_row_gather(D, B):
    info = plsc.get_sparse_core_info()
    NW = info.num_cores * info.num_subcores
    NC = info.num_cores
    assert B % NW == 0
    b_per_w = B // NW
    mesh = plsc.VectorSubcoreMesh(core_axis_name="c", subcore_axis_name="s")

    def k(table_hbm, idx_hbm, out_hbm, idx_v, rows_v, sem):
        wid = lax.axis_index("s") * NC + lax.axis_index("c")
        base = wid * b_per_w
        pltpu.sync_copy(idx_hbm.at[pl.ds(base, b_per_w)], idx_v)
        pltpu.async_copy(table_hbm.at[idx_v], rows_v, sem).wait()
        pltpu.sync_copy(rows_v, out_hbm.at[pl.ds(base, b_per_w)])

    def run(table, idx):
        V = table.shape[0]
        return pl.kernel(
            k, mesh=mesh,
            out_type=jax.ShapeDtypeStruct((B, D), jnp.float32),
            scratch_types=[
                pltpu.VMEM((b_per_w,), jnp.int32),
                pltpu.VMEM((b_per_w, D), jnp.float32),
                pltpu.SemaphoreType.DMA,
            ],
        )(table, idx)

    return run


def _make_row_scatter(D, B, OUT_ROWS):
    info = plsc.get_sparse_core_info()
    NW = info.num_cores * info.num_subcores
    NC = info.num_cores
    assert B % NW == 0
    b_per_w = B // NW
    mesh = plsc.VectorSubcoreMesh(core_axis_name="c", subcore_axis_name="s")

    def k(rows_hbm, idx_hbm, out_hbm, idx0_v, idx1_v, rows_v, sem):
        wid = lax.axis_index("s") * NC + lax.axis_index("c")
        base = wid * b_per_w
        pltpu.sync_copy(rows_hbm.at[pl.ds(base, b_per_w)], rows_v)
        pltpu.sync_copy(idx_hbm.at[pl.ds(base, b_per_w)], idx0_v)
        pltpu.sync_copy(idx_hbm.at[pl.ds(B + base, b_per_w)], idx1_v)
        c0 = pltpu.async_copy(rows_v, out_hbm.at[idx0_v], sem)
        c1 = pltpu.async_copy(rows_v, out_hbm.at[idx1_v], sem)
        c0.wait()
        c1.wait()

    def run(rows, idx):
        return pl.kernel(
            k, mesh=mesh,
            out_type=jax.ShapeDtypeStruct((OUT_ROWS, D), jnp.float32),
            scratch_types=[
                pltpu.VMEM((b_per_w,), jnp.int32),
                pltpu.VMEM((b_per_w,), jnp.int32),
                pltpu.VMEM((b_per_w, D), jnp.float32),
                pltpu.SemaphoreType.DMA,
            ],
        )(rows, idx)

    return run


def _routing_math(h, wg):
    logits = jnp.dot(h, wg, preferred_element_type=jnp.float32)

    eiota = lax.broadcasted_iota(jnp.int32, logits.shape, 1)
    v0 = jnp.max(logits, axis=-1, keepdims=True)
    i0 = jnp.min(jnp.where(logits == v0, eiota, _E), axis=-1, keepdims=True)
    masked = jnp.where(eiota == i0, -jnp.inf, logits)
    v1 = jnp.max(masked, axis=-1, keepdims=True)
    i1 = jnp.min(jnp.where(masked == v1, eiota, _E), axis=-1, keepdims=True)

    ex1 = jnp.exp(v1 - v0)
    w0 = 1.0 / (1.0 + ex1)
    w1 = ex1 / (1.0 + ex1)
    wts = jnp.concatenate([w0, w1], axis=1)

    one0 = (eiota == i0).astype(jnp.bfloat16)
    one1 = (eiota == i1).astype(jnp.bfloat16)

    T = h.shape[0]
    r_iota = lax.broadcasted_iota(jnp.int32, (T, T), 0)
    c_iota = lax.broadcasted_iota(jnp.int32, (T, T), 1)
    tril = (c_iota < r_iota).astype(jnp.bfloat16)

    cum0 = jnp.dot(tril, one0, preferred_element_type=jnp.float32)
    cum1 = jnp.dot(tril, one1, preferred_element_type=jnp.float32)
    tot0 = jnp.sum(one0.astype(jnp.float32), axis=0, keepdims=True)
    tot1 = jnp.sum(one1.astype(jnp.float32), axis=0, keepdims=True)
    counts = tot0 + tot1

    pc = jnp.ceil(counts * (1.0 / _BM)) * _BM
    e_r = lax.broadcasted_iota(jnp.int32, (_E, _E), 0)
    e_c = lax.broadcasted_iota(jnp.int32, (_E, _E), 1)
    m8 = (e_r < e_c).astype(jnp.float32)
    po = jnp.dot(pc, m8, preferred_element_type=jnp.float32)

    rank0 = jnp.sum(one0.astype(jnp.float32) * (cum0 + po), axis=1, keepdims=True)
    rank1 = jnp.sum(one1.astype(jnp.float32) * (cum1 + tot0 + po), axis=1,
                    keepdims=True)
    dest = jnp.concatenate(
        [rank0, rank1], axis=1).astype(jnp.int32)

    pend = po + pc
    bstart = (lax.broadcasted_iota(jnp.int32, (_NB, _E), 0) * _BM).astype(
        jnp.float32)
    be = jnp.sum((jnp.broadcast_to(pend, (_NB, _E)) <= bstart).astype(
        jnp.int32), axis=1, keepdims=True)
    return dest, wts, jnp.minimum(be, _E - 1)


def _router_body(h_ref, wg_ref, dest_ref, wts_ref, be_ref):
    dest, wts, be = _routing_math(h_ref[...], wg_ref[...])
    dest_ref[...] = dest
    wts_ref[...] = wts
    be_ref[...] = be


def _router(h, Wg):
    return pl.pallas_call(
        _router_body,
        in_specs=[
            pl.BlockSpec((_T, _D), lambda: (0, 0)),
            pl.BlockSpec((_D, _E), lambda: (0, 0)),
        ],
        out_specs=[
            pl.BlockSpec((_T, 2), lambda: (0, 0)),
            pl.BlockSpec((_T, 2), lambda: (0, 0)),
            pl.BlockSpec((_NB, 1), lambda: (0, 0)),
        ],
        out_shape=[
            jax.ShapeDtypeStruct((_T, 2), jnp.int32),
            jax.ShapeDtypeStruct((_T, 2), jnp.float32),
            jax.ShapeDtypeStruct((_NB, 1), jnp.int32),
        ],
    )(h, Wg)


def _combine_router_body(a_ref, wts_ref, wg_ref, y_ref, dest_ref, wts2_ref,
                         be_ref):
    a0 = a_ref[0:_T, :]
    a1 = a_ref[_T:2 * _T, :]
    w = wts_ref[...].astype(jnp.bfloat16).astype(jnp.float32)
    y = (w[:, 0:1] * a0 + w[:, 1:2] * a1).astype(jnp.bfloat16)
    y_ref[...] = y.astype(jnp.float32)
    dest, wts2, be = _routing_math(y, wg_ref[...])
    dest_ref[...] = dest
    wts2_ref[...] = wts2
    be_ref[...] = be


def _combine_router(A, wts, Wg):
    return pl.pallas_call(
        _combine_router_body,
        in_specs=[
            pl.BlockSpec((2 * _T, _D), lambda: (0, 0)),
            pl.BlockSpec((_T, 2), lambda: (0, 0)),
            pl.BlockSpec((_D, _E), lambda: (0, 0)),
        ],
        out_specs=[
            pl.BlockSpec((_T, _D), lambda: (0, 0)),
            pl.BlockSpec((_T, 2), lambda: (0, 0)),
            pl.BlockSpec((_T, 2), lambda: (0, 0)),
            pl.BlockSpec((_NB, 1), lambda: (0, 0)),
        ],
        out_shape=[
            jax.ShapeDtypeStruct((_T, _D), jnp.float32),
            jax.ShapeDtypeStruct((_T, 2), jnp.int32),
            jax.ShapeDtypeStruct((_T, 2), jnp.float32),
            jax.ShapeDtypeStruct((_NB, 1), jnp.int32),
        ],
    )(A, wts, Wg)


def _grouped_body(be_ref, x_ref, w1_ref, b1_ref, w2_ref, b2_ref, o_ref):
    x = x_ref[...].astype(jnp.bfloat16)
    hid = jnp.maximum(
        jnp.dot(x, w1_ref[0], preferred_element_type=jnp.float32) + b1_ref[0],
        0.0).astype(jnp.bfloat16)
    out = (jnp.dot(hid, w2_ref[0], preferred_element_type=jnp.float32)
           + b2_ref[0]).astype(jnp.bfloat16)
    o_ref[...] = out.astype(jnp.float32)


def _grouped(Xs, W1, b1, W2, b2, be):
    grid_spec = pltpu.PrefetchScalarGridSpec(
        num_scalar_prefetch=1,
        grid=(_NB,),
        in_specs=[
            pl.BlockSpec((_BM, _D), lambda b, be: (b, 0)),
            pl.BlockSpec((1, _D, _D), lambda b, be: (be[b], 0, 0)),
            pl.BlockSpec((1, 1, _D), lambda b, be: (be[b], 0, 0)),
            pl.BlockSpec((1, _D, _D), lambda b, be: (be[b], 0, 0)),
            pl.BlockSpec((1, 1, _D), lambda b, be: (be[b], 0, 0)),
        ],
        out_specs=pl.BlockSpec((_BM, _D), lambda b, be: (b, 0)),
    )
    return pl.pallas_call(
        _grouped_body,
        grid_spec=grid_spec,
        out_shape=jax.ShapeDtypeStruct((_NPAD, _D), jnp.float32),
    )(be, Xs, W1, b1.reshape(_E, 1, _D), W2, b2.reshape(_E, 1, _D))


def _combine_body(a_ref, wts_ref, o_ref):
    a0 = a_ref[0:_T, :]
    a1 = a_ref[_T:2 * _T, :]
    w = wts_ref[...].astype(jnp.bfloat16).astype(jnp.float32)
    y = w[:, 0:1] * a0 + w[:, 1:2] * a1
    o_ref[...] = y.astype(jnp.bfloat16)


def _combine(A, wts):
    return pl.pallas_call(
        _combine_body,
        in_specs=[
            pl.BlockSpec((2 * _T, _D), lambda: (0, 0)),
            pl.BlockSpec((_T, 2), lambda: (0, 0)),
        ],
        out_specs=pl.BlockSpec((_T, _D), lambda: (0, 0)),
        out_shape=jax.ShapeDtypeStruct((_T, _D), jnp.bfloat16),
    )(A, wts)


def _proj_body(h_ref, w_ref, b_ref, out_ref):
    out_ref[...] = (
        jnp.dot(h_ref[...], w_ref[...], preferred_element_type=jnp.float32)
        + b_ref[...]
    )


def _proj(h, Wout, bout2d, bn=1280):
    T, D = h.shape
    V = Wout.shape[1]
    grid = (V // bn,)
    return pl.pallas_call(
        _proj_body,
        grid=grid,
        in_specs=[
            pl.BlockSpec((T, D), lambda n: (0, 0)),
            pl.BlockSpec((D, bn), lambda n: (0, n)),
            pl.BlockSpec((1, bn), lambda n: (0, n)),
        ],
        out_specs=pl.BlockSpec((T, bn), lambda n: (0, n)),
        out_shape=jax.ShapeDtypeStruct((T, V), jnp.float32),
    )(h, Wout, bout2d)


def _dispatch_ffn(h_f32, dest, be, W1, b1, W2, b2):
    dest_flat = jnp.concatenate([dest[:, 0], dest[:, 1]])
    Xs = _make_row_scatter(_D, _T, _NPAD)(h_f32, dest_flat)
    out_s = _grouped(Xs, W1, b1, W2, b2, be.reshape(_NB))
    return _make_row_gather(_D, 2 * _T)(out_s, dest_flat)


def kernel(x, emb, Wg1, W1a, b1a, W2a, b2a, Wg2, W1b, b1b, W2b, b2b, Wout, bout):
    B, S = x.shape
    bf = jnp.bfloat16
    idx = x.reshape(-1).astype(jnp.int32)
    h32 = _make_row_gather(_D, _T)(emb, idx)
    h_bf = h32.astype(bf)
    h32r = h_bf.astype(jnp.float32)

    dest1, wts1, be1 = _router(h_bf, Wg1.astype(bf))
    A1 = _dispatch_ffn(h32r, dest1, be1, W1a.astype(bf), b1a,
                       W2a.astype(bf), b2a)
    y1, dest2, wts2, be2 = _combine_router(A1, wts1, Wg2.astype(bf))
    A2 = _dispatch_ffn(y1, dest2, be2, W1b.astype(bf), b1b,
                       W2b.astype(bf), b2b)
    h2 = _combine(A2, wts2)
    out = _proj(h2, Wout.astype(bf), bout.reshape(1, -1))
    return out.reshape(B, S, _VOCAB)


# speedup vs baseline: 2.7289x; 1.0148x over previous
import functools

import jax
import jax.numpy as jnp
from jax import lax
from jax.experimental import pallas as pl
from jax.experimental.pallas import tpu as pltpu
from jax.experimental.pallas import tpu_sc as plsc

_VOCAB = 32000
_D = 768
_E = 8
_T = 2048
_BM = 256
_NPAD = _T * 2 + _E * _BM
_NB = _NPAD // _BM


def _make_row_gather(D, B):
    info = plsc.get_sparse_core_info()
    NW = info.num_cores * info.num_subcores
    NC = info.num_cores
    assert B % NW == 0
    b_per_w = B // NW
    mesh = plsc.VectorSubcoreMesh(core_axis_name="c", subcore_axis_name="s")

    def k(table_hbm, idx_hbm, out_hbm, idx_v, rows_v, sem):
        wid = lax.axis_index("s") * NC + lax.axis_index("c")
        base = wid * b_per_w
        pltpu.sync_copy(idx_hbm.at[pl.ds(base, b_per_w)], idx_v)
        pltpu.async_copy(table_hbm.at[idx_v], rows_v, sem).wait()
        pltpu.sync_copy(rows_v, out_hbm.at[pl.ds(base, b_per_w)])

    def run(table, idx):
        V = table.shape[0]
        return pl.kernel(
            k, mesh=mesh,
            out_type=jax.ShapeDtypeStruct((B, D), jnp.float32),
            scratch_types=[
                pltpu.VMEM((b_per_w,), jnp.int32),
                pltpu.VMEM((b_per_w, D), jnp.float32),
                pltpu.SemaphoreType.DMA,
            ],
        )(table, idx)

    return run


def _make_row_scatter(D, B, OUT_ROWS):
    info = plsc.get_sparse_core_info()
    NW = info.num_cores * info.num_subcores
    NC = info.num_cores
    assert B % NW == 0
    b_per_w = B // NW
    mesh = plsc.VectorSubcoreMesh(core_axis_name="c", subcore_axis_name="s")

    def k(rows_hbm, idx_hbm, out_hbm, idx0_v, idx1_v, rows_v, sem):
        wid = lax.axis_index("s") * NC + lax.axis_index("c")
        base = wid * b_per_w
        pltpu.sync_copy(rows_hbm.at[pl.ds(base, b_per_w)], rows_v)
        pltpu.sync_copy(idx_hbm.at[pl.ds(base, b_per_w)], idx0_v)
        pltpu.sync_copy(idx_hbm.at[pl.ds(B + base, b_per_w)], idx1_v)
        c0 = pltpu.async_copy(rows_v, out_hbm.at[idx0_v], sem)
        c1 = pltpu.async_copy(rows_v, out_hbm.at[idx1_v], sem)
        c0.wait()
        c1.wait()

    def run(rows, idx):
        return pl.kernel(
            k, mesh=mesh,
            out_type=jax.ShapeDtypeStruct((OUT_ROWS, D), jnp.float32),
            scratch_types=[
                pltpu.VMEM((b_per_w,), jnp.int32),
                pltpu.VMEM((b_per_w,), jnp.int32),
                pltpu.VMEM((b_per_w, D), jnp.float32),
                pltpu.SemaphoreType.DMA,
            ],
        )(rows, idx)

    return run


def _routing_math(h, wg):
    logits = jnp.dot(h, wg, preferred_element_type=jnp.float32)

    eiota = lax.broadcasted_iota(jnp.int32, logits.shape, 1)
    v0 = jnp.max(logits, axis=-1, keepdims=True)
    i0 = jnp.min(jnp.where(logits == v0, eiota, _E), axis=-1, keepdims=True)
    masked = jnp.where(eiota == i0, -jnp.inf, logits)
    v1 = jnp.max(masked, axis=-1, keepdims=True)
    i1 = jnp.min(jnp.where(masked == v1, eiota, _E), axis=-1, keepdims=True)

    ex1 = jnp.exp(v1 - v0)
    w0 = 1.0 / (1.0 + ex1)
    w1 = ex1 / (1.0 + ex1)
    wts = jnp.concatenate([w0, w1], axis=1)

    one0 = (eiota == i0).astype(jnp.bfloat16)
    one1 = (eiota == i1).astype(jnp.bfloat16)

    T = h.shape[0]
    r_iota = lax.broadcasted_iota(jnp.int32, (T, T), 0)
    c_iota = lax.broadcasted_iota(jnp.int32, (T, T), 1)
    tril = (c_iota < r_iota).astype(jnp.bfloat16)

    cum0 = jnp.dot(tril, one0, preferred_element_type=jnp.float32)
    cum1 = jnp.dot(tril, one1, preferred_element_type=jnp.float32)
    tot0 = jnp.sum(one0.astype(jnp.float32), axis=0, keepdims=True)
    tot1 = jnp.sum(one1.astype(jnp.float32), axis=0, keepdims=True)
    counts = tot0 + tot1

    pc = jnp.ceil(counts * (1.0 / _BM)) * _BM
    e_r = lax.broadcasted_iota(jnp.int32, (_E, _E), 0)
    e_c = lax.broadcasted_iota(jnp.int32, (_E, _E), 1)
    m8 = (e_r < e_c).astype(jnp.float32)
    po = jnp.dot(pc, m8, preferred_element_type=jnp.float32)

    rank0 = jnp.sum(one0.astype(jnp.float32) * (cum0 + po), axis=1, keepdims=True)
    rank1 = jnp.sum(one1.astype(jnp.float32) * (cum1 + tot0 + po), axis=1,
                    keepdims=True)
    dest = jnp.concatenate(
        [rank0, rank1], axis=1).astype(jnp.int32)

    pend = po + pc
    bstart = (lax.broadcasted_iota(jnp.int32, (_NB, _E), 0) * _BM).astype(
        jnp.float32)
    be = jnp.sum((jnp.broadcast_to(pend, (_NB, _E)) <= bstart).astype(
        jnp.int32), axis=1, keepdims=True)
    return dest, wts, jnp.minimum(be, _E - 1)


def _router_body(h_ref, wg_ref, dest_ref, wts_ref, be_ref):
    dest, wts, be = _routing_math(h_ref[...], wg_ref[...])
    dest_ref[...] = dest
    wts_ref[...] = wts
    be_ref[...] = be


def _router(h, Wg):
    return pl.pallas_call(
        _router_body,
        in_specs=[
            pl.BlockSpec((_T, _D), lambda: (0, 0)),
            pl.BlockSpec((_D, _E), lambda: (0, 0)),
        ],
        out_specs=[
            pl.BlockSpec((_T, 2), lambda: (0, 0)),
            pl.BlockSpec((_T, 2), lambda: (0, 0)),
            pl.BlockSpec((_NB, 1), lambda: (0, 0)),
        ],
        out_shape=[
            jax.ShapeDtypeStruct((_T, 2), jnp.int32),
            jax.ShapeDtypeStruct((_T, 2), jnp.float32),
            jax.ShapeDtypeStruct((_NB, 1), jnp.int32),
        ],
    )(h, Wg)


def _combine_router_body(a_ref, wts_ref, wg_ref, y_ref, dest_ref, wts2_ref,
                         be_ref):
    a0 = a_ref[0:_T, :]
    a1 = a_ref[_T:2 * _T, :]
    w = wts_ref[...].astype(jnp.bfloat16).astype(jnp.float32)
    y = (w[:, 0:1] * a0 + w[:, 1:2] * a1).astype(jnp.bfloat16)
    y_ref[...] = y.astype(jnp.float32)
    dest, wts2, be = _routing_math(y, wg_ref[...])
    dest_ref[...] = dest
    wts2_ref[...] = wts2
    be_ref[...] = be


def _combine_router(A, wts, Wg):
    return pl.pallas_call(
        _combine_router_body,
        in_specs=[
            pl.BlockSpec((2 * _T, _D), lambda: (0, 0)),
            pl.BlockSpec((_T, 2), lambda: (0, 0)),
            pl.BlockSpec((_D, _E), lambda: (0, 0)),
        ],
        out_specs=[
            pl.BlockSpec((_T, _D), lambda: (0, 0)),
            pl.BlockSpec((_T, 2), lambda: (0, 0)),
            pl.BlockSpec((_T, 2), lambda: (0, 0)),
            pl.BlockSpec((_NB, 1), lambda: (0, 0)),
        ],
        out_shape=[
            jax.ShapeDtypeStruct((_T, _D), jnp.float32),
            jax.ShapeDtypeStruct((_T, 2), jnp.int32),
            jax.ShapeDtypeStruct((_T, 2), jnp.float32),
            jax.ShapeDtypeStruct((_NB, 1), jnp.int32),
        ],
    )(A, wts, Wg)


def _grouped_body(be_ref, x_ref, w1_ref, b1_ref, w2_ref, b2_ref, o_ref):
    x = x_ref[...].astype(jnp.bfloat16)
    hid = jnp.maximum(
        jnp.dot(x, w1_ref[0], preferred_element_type=jnp.float32) + b1_ref[0],
        0.0).astype(jnp.bfloat16)
    out = (jnp.dot(hid, w2_ref[0], preferred_element_type=jnp.float32)
           + b2_ref[0]).astype(jnp.bfloat16)
    o_ref[...] = out.astype(jnp.float32)


def _grouped(Xs, W1, b1, W2, b2, be):
    grid_spec = pltpu.PrefetchScalarGridSpec(
        num_scalar_prefetch=1,
        grid=(_NB,),
        in_specs=[
            pl.BlockSpec((_BM, _D), lambda b, be: (b, 0)),
            pl.BlockSpec((1, _D, _D), lambda b, be: (be[b], 0, 0)),
            pl.BlockSpec((1, 1, _D), lambda b, be: (be[b], 0, 0)),
            pl.BlockSpec((1, _D, _D), lambda b, be: (be[b], 0, 0)),
            pl.BlockSpec((1, 1, _D), lambda b, be: (be[b], 0, 0)),
        ],
        out_specs=pl.BlockSpec((_BM, _D), lambda b, be: (b, 0)),
    )
    return pl.pallas_call(
        _grouped_body,
        grid_spec=grid_spec,
        out_shape=jax.ShapeDtypeStruct((_NPAD, _D), jnp.float32),
    )(be, Xs, W1, b1.reshape(_E, 1, _D), W2, b2.reshape(_E, 1, _D))


def _combine_body(a_ref, wts_ref, o_ref):
    a0 = a_ref[0:_T, :]
    a1 = a_ref[_T:2 * _T, :]
    w = wts_ref[...].astype(jnp.bfloat16).astype(jnp.float32)
    y = w[:, 0:1] * a0 + w[:, 1:2] * a1
    o_ref[...] = y.astype(jnp.bfloat16)


def _combine(A, wts):
    return pl.pallas_call(
        _combine_body,
        in_specs=[
            pl.BlockSpec((2 * _T, _D), lambda: (0, 0)),
            pl.BlockSpec((_T, 2), lambda: (0, 0)),
        ],
        out_specs=pl.BlockSpec((_T, _D), lambda: (0, 0)),
        out_shape=jax.ShapeDtypeStruct((_T, _D), jnp.bfloat16),
    )(A, wts)


def _proj_body(h_ref, w_ref, b_ref, out_ref):
    out_ref[...] = (
        jnp.dot(h_ref[...], w_ref[...], preferred_element_type=jnp.float32)
        + b_ref[...]
    )


def _proj(h, Wout, bout2d, bn=2560):
    T, D = h.shape
    V = Wout.shape[1]
    grid = (V // bn,)
    return pl.pallas_call(
        _proj_body,
        grid=grid,
        in_specs=[
            pl.BlockSpec((T, D), lambda n: (0, 0)),
            pl.BlockSpec((D, bn), lambda n: (0, n)),
            pl.BlockSpec((1, bn), lambda n: (0, n)),
        ],
        out_specs=pl.BlockSpec((T, bn), lambda n: (0, n)),
        out_shape=jax.ShapeDtypeStruct((T, V), jnp.float32),
    )(h, Wout, bout2d)


def _dispatch_ffn(h_f32, dest, be, W1, b1, W2, b2):
    dest_flat = jnp.concatenate([dest[:, 0], dest[:, 1]])
    Xs = _make_row_scatter(_D, _T, _NPAD)(h_f32, dest_flat)
    out_s = _grouped(Xs, W1, b1, W2, b2, be.reshape(_NB))
    return _make_row_gather(_D, 2 * _T)(out_s, dest_flat)


def kernel(x, emb, Wg1, W1a, b1a, W2a, b2a, Wg2, W1b, b1b, W2b, b2b, Wout, bout):
    B, S = x.shape
    bf = jnp.bfloat16
    idx = x.reshape(-1).astype(jnp.int32)
    h32 = _make_row_gather(_D, _T)(emb, idx)
    h_bf = h32.astype(bf)
    h32r = h_bf.astype(jnp.float32)

    dest1, wts1, be1 = _router(h_bf, Wg1.astype(bf))
    A1 = _dispatch_ffn(h32r, dest1, be1, W1a.astype(bf), b1a,
                       W2a.astype(bf), b2a)
    y1, dest2, wts2, be2 = _combine_router(A1, wts1, Wg2.astype(bf))
    A2 = _dispatch_ffn(y1, dest2, be2, W1b.astype(bf), b1b,
                       W2b.astype(bf), b2b)
    h2 = _combine(A2, wts2)
    out = _proj(h2, Wout.astype(bf), bout.reshape(1, -1))
    return out.reshape(B, S, _VOCAB)
